# baseline (speedup 1.0000x reference)
import functools

import jax
import jax.numpy as jnp
from jax import lax
from jax.experimental import pallas as pl
from jax.experimental.pallas import tpu as pltpu
from jax.experimental.pallas import tpu_sc as plsc

N_NODES = 10000
N_EDGES = 320000
D = 128
H = D // 2
DW = D // 2

NC = 2
NS = 16
NW = NC * NS
SLABS = 2
SLAB = N_EDGES // SLABS
EPW = SLAB // NW
CH = 40
NCH = EPW // CH
PIPE = 5


def _make_gather_mul(slab):
    src_off = slab * SLAB
    dst_off = N_EDGES + slab * SLAB
    mesh = plsc.VectorSubcoreMesh(core_axis_name="c", subcore_axis_name="s")

    @functools.partial(
        pl.kernel,
        out_type=jax.ShapeDtypeStruct((SLAB, DW), jnp.float32),
        mesh=mesh,
        compiler_params=pltpu.CompilerParams(needs_layout_passes=False),
        scratch_types=[
            pltpu.VMEM((EPW,), jnp.int32),
            pltpu.VMEM((EPW,), jnp.int32),
            [pltpu.VMEM((CH, D), jnp.float32) for _ in range(PIPE)],
            [pltpu.VMEM((CH, D), jnp.float32) for _ in range(PIPE)],
            [pltpu.VMEM((CH, DW), jnp.float32) for _ in range(PIPE)],
            [pltpu.SemaphoreType.DMA for _ in range(PIPE)],
            [pltpu.SemaphoreType.DMA for _ in range(PIPE)],
            [pltpu.SemaphoreType.DMA for _ in range(PIPE)],
        ],
    )
    def gather_mul(src_hbm, dst_hbm, eidx_hbm, out_hbm,
                   sidx_v, didx_v, srows, drows, orows, sem_s, sem_d, sem_o):
        wid = lax.axis_index("s") * NC + lax.axis_index("c")
        base = wid * EPW
        pltpu.sync_copy(eidx_hbm.at[pl.ds(src_off + base, EPW)], sidx_v)
        pltpu.sync_copy(eidx_hbm.at[pl.ds(dst_off + base, EPW)], didx_v)

        def fire_gather(c, b):
            pltpu.async_copy(src_hbm.at[sidx_v.at[pl.ds(c * CH, CH)]],
                             srows[b], sem_s[b])
            pltpu.async_copy(dst_hbm.at[didx_v.at[pl.ds(c * CH, CH)]],
                             drows[b], sem_d[b])

        def wait_gather(b):
            pltpu.make_async_copy(src_hbm.at[sidx_v.at[pl.ds(0, CH)]],
                                  srows[b], sem_s[b]).wait()
            pltpu.make_async_copy(dst_hbm.at[didx_v.at[pl.ds(0, CH)]],
                                  drows[b], sem_d[b]).wait()

        for p in range(PIPE):
            fire_gather(p, p)

        def round_body(k, carry):
            for b in range(PIPE):
                c = PIPE * k + b
                wait_gather(b)

                @pl.when(c >= PIPE)
                def _wait_prev_out():
                    pltpu.make_async_copy(
                        orows[b], out_hbm.at[pl.ds(base, CH)], sem_o[b]).wait()

                @plsc.parallel_loop(0, CH, 1, unroll=4)
                def _row_body(r):
                    fmt = plsc.PackFormat.INTERLEAVED
                    for j in range(DW // 16):
                        s0 = srows[b][r, pl.ds(2 * j * 16, 16)]
                        s1 = srows[b][r, pl.ds((2 * j + 1) * 16, 16)]
                        d0 = drows[b][r, pl.ds(2 * j * 16, 16)]
                        d1 = drows[b][r, pl.ds((2 * j + 1) * 16, 16)]
                        p = plsc.pack(s0 * d0, s1 * d1, format=fmt)
                        orows[b][r, pl.ds(j * 16, 16)] = plsc.bitcast(
                            p, jnp.float32)

                pltpu.async_copy(orows[b],
                                 out_hbm.at[pl.ds(base + c * CH, CH)],
                                 sem_o[b])

                @pl.when(c + PIPE < NCH)
                def _prefetch():
                    fire_gather(c + PIPE, b)
            return carry

        lax.fori_loop(0, NCH // PIPE, round_body, 0)
        for b in range(PIPE):
            pltpu.make_async_copy(
                orows[b], out_hbm.at[pl.ds(base, CH)], sem_o[b]).wait()

    return gather_mul


_gather_muls = [_make_gather_mul(s) for s in range(SLABS)]

BLK = 8000
NBLK = SLAB // BLK


def _mlp_body(x_ref, w1lo_ref, w1hi_ref, b1_ref, w2t_ref, b2_ref, o_ref):
    xi = jax.lax.bitcast_convert_type(x_ref[...], jnp.int32)
    xlo = jax.lax.bitcast_convert_type(xi << 16, jnp.float32)
    xhi = jax.lax.bitcast_convert_type(xi & jnp.int32(-65536), jnp.float32)
    h = (jnp.dot(xlo, w1lo_ref[...], preferred_element_type=jnp.float32)
         + jnp.dot(xhi, w1hi_ref[...], preferred_element_type=jnp.float32))
    h = jnp.maximum(h + b1_ref[...], 0.0)
    y = jax.lax.dot_general(w2t_ref[...], h, (((1,), (1,)), ((), ())),
                            preferred_element_type=jnp.float32)
    o_ref[pl.ds(pl.program_id(0), 1), :] = jax.nn.sigmoid(y + b2_ref[...])


def _mlp(x, W1lo, W1hi, b1, W2t, b2):
    grid = (NBLK,)
    return pl.pallas_call(
        _mlp_body,
        grid=grid,
        in_specs=[
            pl.BlockSpec((BLK, DW), lambda i: (i, 0)),
            pl.BlockSpec((DW, H), lambda i: (0, 0)),
            pl.BlockSpec((DW, H), lambda i: (0, 0)),
            pl.BlockSpec((1, H), lambda i: (0, 0)),
            pl.BlockSpec((1, H), lambda i: (0, 0)),
            pl.BlockSpec((1, 1), lambda i: (0, 0)),
        ],
        out_specs=pl.BlockSpec((NBLK, BLK), lambda i: (0, 0)),
        out_shape=jax.ShapeDtypeStruct((NBLK, BLK), jnp.float32),
    )(x, W1lo, W1hi, b1, W2t, b2)


def _pack_perms():
    lo, hi = [], []
    for j in range(D // 32):
        for w in range(16):
            lo.append(32 * j + w)
            hi.append(32 * j + 16 + w)
    return (jnp.asarray(lo, dtype=jnp.int32), jnp.asarray(hi, dtype=jnp.int32))


def kernel(dst_embs, src_embs, edge_indices, W1, b1, W2, b2):
    eidx = edge_indices.reshape(2 * N_EDGES)
    perm_lo, perm_hi = _pack_perms()
    w1lo = W1[perm_lo, :]
    w1hi = W1[perm_hi, :]
    b1r, w2t, b2r = b1.reshape(1, H), W2.reshape(1, H), b2.reshape(1, 1)
    ys = []
    for s in range(SLABS):
        x = _gather_muls[s](src_embs, dst_embs, eidx)
        ys.append(_mlp(x, w1lo, w1hi, b1r, w2t, b2r))
    y = jnp.concatenate(ys, axis=0)
    return y.reshape(N_EDGES, 1)


# speedup vs baseline: 1.0177x; 1.0177x over previous
import functools

import jax
import jax.numpy as jnp
from jax import lax
from jax.experimental import pallas as pl
from jax.experimental.pallas import tpu as pltpu
from jax.experimental.pallas import tpu_sc as plsc

N_NODES = 10000
N_EDGES = 320000
D = 128
H = D // 2
DW = D // 2

NC = 2
NS = 16
NW = NC * NS
SLABS = 2
SLAB = N_EDGES // SLABS
EPW = SLAB // NW
CH = 40
NCH = EPW // CH
PIPE = 5


def _make_gather_mul(slab):
    src_off = slab * SLAB
    dst_off = N_EDGES + slab * SLAB
    mesh = plsc.VectorSubcoreMesh(core_axis_name="c", subcore_axis_name="s")

    @functools.partial(
        pl.kernel,
        out_type=jax.ShapeDtypeStruct((SLAB, DW), jnp.float32),
        mesh=mesh,
        compiler_params=pltpu.CompilerParams(needs_layout_passes=False),
        scratch_types=[
            pltpu.VMEM((EPW,), jnp.int32),
            pltpu.VMEM((EPW,), jnp.int32),
            [pltpu.VMEM((CH, D), jnp.float32) for _ in range(PIPE)],
            [pltpu.VMEM((CH, D), jnp.float32) for _ in range(PIPE)],
            [pltpu.VMEM((CH, DW), jnp.float32) for _ in range(PIPE)],
            [pltpu.SemaphoreType.DMA for _ in range(PIPE)],
            [pltpu.SemaphoreType.DMA for _ in range(PIPE)],
            [pltpu.SemaphoreType.DMA for _ in range(PIPE)],
        ],
    )
    def gather_mul(src_hbm, dst_hbm, eidx_hbm, out_hbm,
                   sidx_v, didx_v, srows, drows, orows, sem_s, sem_d, sem_o):
        wid = lax.axis_index("s") * NC + lax.axis_index("c")
        base = wid * EPW
        pltpu.sync_copy(eidx_hbm.at[pl.ds(src_off + base, EPW)], sidx_v)
        pltpu.sync_copy(eidx_hbm.at[pl.ds(dst_off + base, EPW)], didx_v)

        def fire_gather(c, b):
            pltpu.async_copy(src_hbm.at[sidx_v.at[pl.ds(c * CH, CH)]],
                             srows[b], sem_s[b])
            pltpu.async_copy(dst_hbm.at[didx_v.at[pl.ds(c * CH, CH)]],
                             drows[b], sem_d[b])

        def wait_gather(b):
            pltpu.make_async_copy(src_hbm.at[sidx_v.at[pl.ds(0, CH)]],
                                  srows[b], sem_s[b]).wait()
            pltpu.make_async_copy(dst_hbm.at[didx_v.at[pl.ds(0, CH)]],
                                  drows[b], sem_d[b]).wait()

        for p in range(PIPE):
            fire_gather(p, p)

        def round_body(k, carry):
            for b in range(PIPE):
                c = PIPE * k + b
                wait_gather(b)

                @pl.when(c >= PIPE)
                def _wait_prev_out():
                    pltpu.make_async_copy(
                        orows[b], out_hbm.at[pl.ds(base, CH)], sem_o[b]).wait()

                @plsc.parallel_loop(0, CH, 1, unroll=4)
                def _row_body(r):
                    fmt = plsc.PackFormat.INTERLEAVED
                    for j in range(DW // 16):
                        s0 = srows[b][r, pl.ds(2 * j * 16, 16)]
                        s1 = srows[b][r, pl.ds((2 * j + 1) * 16, 16)]
                        d0 = drows[b][r, pl.ds(2 * j * 16, 16)]
                        d1 = drows[b][r, pl.ds((2 * j + 1) * 16, 16)]
                        p = plsc.pack(s0 * d0, s1 * d1, format=fmt)
                        orows[b][r, pl.ds(j * 16, 16)] = plsc.bitcast(
                            p, jnp.float32)

                pltpu.async_copy(orows[b],
                                 out_hbm.at[pl.ds(base + c * CH, CH)],
                                 sem_o[b])

                @pl.when(c + PIPE < NCH)
                def _prefetch():
                    fire_gather(c + PIPE, b)
            return carry

        lax.fori_loop(0, NCH // PIPE, round_body, 0)
        for b in range(PIPE):
            pltpu.make_async_copy(
                orows[b], out_hbm.at[pl.ds(base, CH)], sem_o[b]).wait()

    return gather_mul


_gather_muls = [_make_gather_mul(s) for s in range(SLABS)]

BLK = 8000
NBLK = SLAB // BLK


def _mlp_body(x_ref, w1c_ref, b1_ref, w2t_ref, b2_ref, o_ref):
    xi = jax.lax.bitcast_convert_type(x_ref[...], jnp.int32)
    xlo = jax.lax.bitcast_convert_type(xi << 16, jnp.float32)
    xhi = jax.lax.bitcast_convert_type(xi & jnp.int32(-65536), jnp.float32)
    xc = jnp.concatenate([xlo, xhi], axis=1).astype(jnp.bfloat16)
    h = jnp.dot(xc, w1c_ref[...], preferred_element_type=jnp.float32)
    h = jnp.maximum(h + b1_ref[...], 0.0)
    y = jax.lax.dot_general(w2t_ref[...], h, (((1,), (1,)), ((), ())),
                            preferred_element_type=jnp.float32)
    o_ref[pl.ds(pl.program_id(0), 1), :] = jax.nn.sigmoid(y + b2_ref[...])


def _mlp(x, W1c, b1, W2t, b2):
    grid = (NBLK,)
    return pl.pallas_call(
        _mlp_body,
        grid=grid,
        in_specs=[
            pl.BlockSpec((BLK, DW), lambda i: (i, 0)),
            pl.BlockSpec((D, H), lambda i: (0, 0)),
            pl.BlockSpec((1, H), lambda i: (0, 0)),
            pl.BlockSpec((1, H), lambda i: (0, 0)),
            pl.BlockSpec((1, 1), lambda i: (0, 0)),
        ],
        out_specs=pl.BlockSpec((NBLK, BLK), lambda i: (0, 0)),
        out_shape=jax.ShapeDtypeStruct((NBLK, BLK), jnp.float32),
    )(x, W1c, b1, W2t, b2)


def _pack_perms():
    lo, hi = [], []
    for j in range(D // 32):
        for w in range(16):
            lo.append(32 * j + w)
            hi.append(32 * j + 16 + w)
    return (jnp.asarray(lo, dtype=jnp.int32), jnp.asarray(hi, dtype=jnp.int32))


def kernel(dst_embs, src_embs, edge_indices, W1, b1, W2, b2):
    eidx = edge_indices.reshape(2 * N_EDGES)
    perm_lo, perm_hi = _pack_perms()
    w1c = jnp.concatenate([W1[perm_lo, :], W1[perm_hi, :]],
                          axis=0).astype(jnp.bfloat16)
    b1r, w2t, b2r = b1.reshape(1, H), W2.reshape(1, H), b2.reshape(1, 1)
    ys = []
    for s in range(SLABS):
        x = _gather_muls[s](src_embs, dst_embs, eidx)
        ys.append(_mlp(x, w1c, b1r, w2t, b2r))
    y = jnp.concatenate(ys, axis=0)
    return y.reshape(N_EDGES, 1)
